# baseline (speedup 1.0000x reference)
import jax
import jax.numpy as jnp
from jax.experimental import pallas as pl
from jax.experimental.pallas import tpu as pltpu

_N_ITERS = 20
_STRIP = 256
_NUM_CORES = 2


def _alloc_body(o_ref):
    pass


def _one_matrix(x_hbm, o_hbm, b, e_ref, in_sem, out_sem):
    n = e_ref.shape[0]
    ns = n // _STRIP

    cp_in = pltpu.make_async_copy(x_hbm.at[b], e_ref, in_sem)
    cp_in.start()
    cp_in.wait()

    def sweep0(s, c_acc):
        rows = pl.ds(s * _STRIP, _STRIP)
        z = e_ref[rows, :]
        m = jnp.max(z, axis=1, keepdims=True)
        e = jnp.exp(z - m)
        r = jnp.sum(e, axis=1, keepdims=True)
        g = e * (1.0 / r)
        e_ref[rows, :] = g
        return c_acc + jnp.sum(g, axis=0, keepdims=True)

    c = jax.lax.fori_loop(0, ns, sweep0, jnp.zeros((1, n), jnp.float32))

    def iter_body(_, c):
        rc = 1.0 / c

        def sweep(s, c_acc):
            rows = pl.ds(s * _STRIP, _STRIP)
            f = e_ref[rows, :] * rc
            r = jnp.sum(f, axis=1, keepdims=True)
            g = f * (1.0 / r)
            e_ref[rows, :] = g
            return c_acc + jnp.sum(g, axis=0, keepdims=True)

        return jax.lax.fori_loop(0, ns, sweep, jnp.zeros((1, n), jnp.float32))

    c = jax.lax.fori_loop(0, _N_ITERS - 1, iter_body, c)

    rc = 1.0 / c

    def final_sweep(s, carry):
        rows = pl.ds(s * _STRIP, _STRIP)
        e_ref[rows, :] = e_ref[rows, :] * rc
        return carry

    jax.lax.fori_loop(0, ns, final_sweep, 0)

    cp_out = pltpu.make_async_copy(e_ref, o_hbm.at[b], out_sem)
    cp_out.start()
    cp_out.wait()


def kernel(log_alpha):
    batch, n, _ = log_alpha.shape
    per_core = batch // _NUM_CORES
    mesh = pltpu.create_tensorcore_mesh("core", num_cores=_NUM_CORES)

    out_init = pl.pallas_call(
        _alloc_body,
        out_shape=jax.ShapeDtypeStruct((batch, n, n), jnp.float32),
        out_specs=pl.BlockSpec(memory_space=pl.ANY),
        name="sinkhorn_out_alloc",
    )()

    def run(refs):
        x_hbm, o_hbm = refs

        @pl.core_map(
            mesh,
            scratch_shapes=[
                pltpu.VMEM((n, n), jnp.float32),
                pltpu.SemaphoreType.DMA,
                pltpu.SemaphoreType.DMA,
            ],
            name="sinkhorn_prob_domain",
        )
        def _(e_ref, in_sem, out_sem):
            core = jax.lax.axis_index("core")

            def per_batch(j, carry):
                _one_matrix(x_hbm, o_hbm, j, e_ref, in_sem, out_sem)
                return carry

            nwork = jnp.where(core == 0, per_core * _NUM_CORES, 0)
            jax.lax.fori_loop(0, nwork, per_batch, 0)

    _, out = pl.run_state(run)((log_alpha, out_init))
    return out


# speedup vs baseline: 1.0189x; 1.0189x over previous
import jax
import jax.numpy as jnp
from jax.experimental import pallas as pl
from jax.experimental.pallas import tpu as pltpu

_N_ITERS = 20
_STRIP = 256


def _normalize_in_place(e_ref, n):
    ns = n // _STRIP

    def sweep0(s, c_acc):
        rows = pl.ds(s * _STRIP, _STRIP)
        z = e_ref[rows, :]
        m = jnp.max(z, axis=1, keepdims=True)
        e = jnp.exp(z - m)
        r = jnp.sum(e, axis=1, keepdims=True)
        g = e * (1.0 / r)
        e_ref[rows, :] = g
        return c_acc + jnp.sum(g, axis=0, keepdims=True)

    c = jax.lax.fori_loop(0, ns, sweep0, jnp.zeros((1, n), jnp.float32))

    def iter_body(_, c):
        rc = 1.0 / c

        def sweep(s, c_acc):
            rows = pl.ds(s * _STRIP, _STRIP)
            f = e_ref[rows, :] * rc
            r = jnp.sum(f, axis=1, keepdims=True)
            g = f * (1.0 / r)
            e_ref[rows, :] = g
            return c_acc + jnp.sum(g, axis=0, keepdims=True)

        return jax.lax.fori_loop(0, ns, sweep, jnp.zeros((1, n), jnp.float32))

    c = jax.lax.fori_loop(0, _N_ITERS - 1, iter_body, c)

    rc = 1.0 / c

    def final_sweep(s, carry):
        rows = pl.ds(s * _STRIP, _STRIP)
        e_ref[rows, :] = e_ref[rows, :] * rc
        return carry

    jax.lax.fori_loop(0, ns, final_sweep, 0)


def _sinkhorn_body(x_hbm, o_hbm, e_scr, in_sems, out_sems):
    i = pl.program_id(0)
    nb = pl.num_programs(0)
    n = e_scr.shape[1]
    buf = jax.lax.rem(i, 2)
    other = 1 - buf

    @pl.when(i == 0)
    def _():
        pltpu.make_async_copy(x_hbm.at[0], e_scr.at[0], in_sems.at[0]).start()
        pltpu.make_async_copy(x_hbm.at[1], e_scr.at[1], in_sems.at[1]).start()

    pltpu.make_async_copy(x_hbm.at[i], e_scr.at[buf], in_sems.at[buf]).wait()

    _normalize_in_place(e_scr.at[buf], n)

    pltpu.make_async_copy(e_scr.at[buf], o_hbm.at[i], out_sems.at[buf]).start()

    @pl.when(jnp.logical_and(i >= 1, i < nb - 1))
    def _():
        pltpu.make_async_copy(e_scr.at[other], o_hbm.at[i - 1],
                              out_sems.at[other]).wait()
        pltpu.make_async_copy(x_hbm.at[i + 1], e_scr.at[other],
                              in_sems.at[other]).start()

    @pl.when(i == nb - 1)
    def _():
        pltpu.make_async_copy(e_scr.at[other], o_hbm.at[i - 1],
                              out_sems.at[other]).wait()
        pltpu.make_async_copy(e_scr.at[buf], o_hbm.at[i],
                              out_sems.at[buf]).wait()


def kernel(log_alpha):
    batch, n, _ = log_alpha.shape
    return pl.pallas_call(
        _sinkhorn_body,
        out_shape=jax.ShapeDtypeStruct((batch, n, n), jnp.float32),
        grid=(batch,),
        in_specs=[pl.BlockSpec(memory_space=pl.ANY)],
        out_specs=pl.BlockSpec(memory_space=pl.ANY),
        scratch_shapes=[
            pltpu.VMEM((2, n, n), jnp.float32),
            pltpu.SemaphoreType.DMA((2,)),
            pltpu.SemaphoreType.DMA((2,)),
        ],
        compiler_params=pltpu.CompilerParams(
            dimension_semantics=("arbitrary",),
            vmem_limit_bytes=48 * 1024 * 1024,
        ),
        name="sinkhorn_prob_domain",
    )(log_alpha)
